# baseline (speedup 1.0000x reference)
import functools

import jax
import jax.numpy as jnp
from jax import lax
from jax.experimental import pallas as pl
from jax.experimental.pallas import tpu as pltpu
from jax.experimental.pallas import tpu_sc as plsc

D = 64
SEQ = 2048
B_TOTAL = 8192
NC, NS, L = 2, 16, 16
NW = NC * NS
BPW = B_TOTAL // NW
CHUNK = 128
NCHUNK = BPW // CHUNK

_mesh = plsc.VectorSubcoreMesh(core_axis_name="c", subcore_axis_name="s")


@functools.partial(
    pl.kernel,
    mesh=_mesh,
    out_type=jax.ShapeDtypeStruct((B_TOTAL, D), jnp.float32),
    scratch_types=[
        pltpu.VMEM((NCHUNK, CHUNK), jnp.int32),
        pltpu.VMEM((BPW, D), jnp.float32),
        pltpu.VMEM((BPW, D), jnp.float32),
        pltpu.SemaphoreType.DMA,
    ],
    compiler_params=pltpu.CompilerParams(use_tc_tiling_on_sc=False),
)
def _emb_lookup(idx_hbm, tok_hbm, pos_hbm, out_hbm, idx_v, rows_v, pos_v, sem):
    wid = lax.axis_index("s") * NC + lax.axis_index("c")
    base = wid * BPW
    pos_base = lax.rem(base, SEQ)

    pltpu.sync_copy(idx_hbm.at[pl.ds(wid * NCHUNK, NCHUNK)], idx_v)

    copies = [
        pltpu.async_copy(
            tok_hbm.at[idx_v.at[j]],
            rows_v.at[pl.ds(j * CHUNK, CHUNK)],
            sem,
        )
        for j in range(NCHUNK)
    ]
    pltpu.sync_copy(pos_hbm.at[pl.ds(pos_base, BPW)], pos_v)
    for cp in copies:
        cp.wait()

    def add_row(r, carry):
        for c in range(D // L):
            sl = pl.ds(c * L, L)
            rows_v[r, sl] = rows_v[r, sl] + pos_v[r, sl]
        return carry

    lax.fori_loop(0, BPW, add_row, 0)

    pltpu.sync_copy(rows_v, out_hbm.at[pl.ds(base, BPW)])


def kernel(x, token_table, pos_table):
    batch, seq = x.shape
    idx = x.astype(jnp.int32).reshape(NW * NCHUNK, CHUNK)
    out = _emb_lookup(idx, token_table, pos_table)
    return out.reshape(batch, seq, D)


# speedup vs baseline: 2.4689x; 2.4689x over previous
import functools

import jax
import jax.numpy as jnp
from jax import lax
from jax.experimental import pallas as pl
from jax.experimental.pallas import tpu as pltpu
from jax.experimental.pallas import tpu_sc as plsc

D = 64
SEQ = 2048
B_TOTAL = 8192
NC, NS, L = 2, 16, 16
NW = NC * NS
BPW = B_TOTAL // NW
NG = BPW // L
BLK = 8

_mesh = plsc.VectorSubcoreMesh(core_axis_name="c", subcore_axis_name="s")


@functools.partial(
    pl.kernel,
    mesh=_mesh,
    out_type=jax.ShapeDtypeStruct((B_TOTAL, D), jnp.float32),
    scratch_types=[
        pltpu.VMEM((NW * 2, 128), jnp.int32),
        pltpu.VMEM((BPW, D), jnp.float32),
        pltpu.VMEM((BPW, D), jnp.float32),
        pltpu.SemaphoreType.DMA,
    ],
)
def _emb_lookup(idx_hbm, tok_hbm, pos_hbm, out_hbm, idx_v, acc, pos_v, sem):
    wid = lax.axis_index("s") * NC + lax.axis_index("c")
    base = wid * BPW
    pos_base = lax.rem(base, SEQ)

    pltpu.sync_copy(idx_hbm, idx_v)

    row0 = wid * 2

    def fire_group(g):
        iv = idx_v[row0 + g // 8, pl.ds((g % 8) * L, L)]
        bv = lax.shift_right_logical(iv, 3)
        sv = lax.bitwise_and(iv, 7)
        return [
            pltpu.async_copy(
                tok_hbm.at[bv[l], sv[l]], acc.at[g * L + l], sem
            )
            for l in range(L)
        ]

    copies = [fire_group(0), fire_group(1)]
    pltpu.sync_copy(pos_hbm.at[pl.ds(pos_base, BPW)], pos_v)
    for g in range(NG):
        if g + 2 < NG:
            copies.append(fire_group(g + 2))
        for cp in copies[g]:
            cp.wait()

    def add_row(r, carry):
        for c in range(D // L):
            sl = pl.ds(c * L, L)
            plsc.addupdate(acc.at[r, sl], pos_v[r, sl])
        return carry

    lax.fori_loop(0, BPW, add_row, 0)

    pltpu.sync_copy(acc, out_hbm.at[pl.ds(base, BPW)])


def kernel(x, token_table, pos_table):
    batch, seq = x.shape
    idx = x.astype(jnp.int32).reshape(NW * 2, 128)
    tok3 = token_table.reshape(-1, BLK, D)
    out = _emb_lookup(idx, tok3, pos_table)
    return out.reshape(batch, seq, D)


# speedup vs baseline: 4.8345x; 1.9581x over previous
import functools

import jax
import jax.numpy as jnp
from jax import lax
from jax.experimental import pallas as pl
from jax.experimental.pallas import tpu as pltpu
from jax.experimental.pallas import tpu_sc as plsc

D = 64
SEQ = 2048
B_TOTAL = 8192
NC, NS, L = 2, 16, 16
NW = NC * NS
BPW = B_TOTAL // NW
GRP = 4
NGRP = BPW // GRP
NBUF = 8

_mesh = plsc.VectorSubcoreMesh(core_axis_name="c", subcore_axis_name="s")


@functools.partial(
    pl.kernel,
    mesh=_mesh,
    out_type=jax.ShapeDtypeStruct((D, B_TOTAL), jnp.float32),
    scratch_types=[
        pltpu.VMEM((NW * 2, 128), jnp.int32),
        pltpu.VMEM((NBUF, D, 128), jnp.float32),
        pltpu.VMEM((2, D, 128), jnp.float32),
        pltpu.SemaphoreType.DMA,
    ],
    compiler_params=pltpu.CompilerParams(needs_layout_passes=False),
)
def _emb_lookup(idx_hbm, tok_hbm, pos_hbm, out_hbm, idx_v, stage, acc, sem):
    wid = lax.axis_index("s") * NC + lax.axis_index("c")
    base = pl.multiple_of(wid * BPW, BPW)
    pos_base = pl.multiple_of(lax.rem(base, SEQ), BPW)

    pltpu.sync_copy(idx_hbm, idx_v)
    for h in range(2):
        pltpu.sync_copy(
            pos_hbm.at[:, pl.ds(pos_base + h * 128, 128)], acc.at[h]
        )

    row0 = wid * 2
    lanes = lax.iota(jnp.int32, L)

    def fire(cv, l, slot):
        col = pl.multiple_of(cv[l] * 128, 128)
        pltpu.async_copy(
            tok_hbm.at[:, pl.ds(col, 128)], stage.at[slot], sem
        )

    def select(pv, l, slot, r):
        pltpu.make_async_copy(
            tok_hbm.at[:, pl.ds(0, 128)], stage.at[slot], sem
        ).wait()
        pvec = jnp.full((L,), pv[l], jnp.int32)
        rvec = jnp.full((L,), lax.rem(r, 128), jnp.int32)
        buf = stage.at[slot]
        half = acc.at[r // 128]
        for q in range(D // L):
            dvec = lanes + (q * L)
            vals = plsc.load_gather(buf, [dvec, pvec])
            plsc.addupdate_scatter(half, [dvec, rvec], vals)

    def step(sg, carry):
        iv = idx_v[row0 + sg // 8, pl.ds(lax.rem(sg, 8) * L, L)]
        cv = lax.shift_right_logical(iv, 7)
        pv = lax.bitwise_and(iv, 127)
        for l in range(NBUF):
            fire(cv, l, l)
        for l in range(NBUF):
            select(pv, l, l, sg * L + l)
            fire(cv, l + NBUF, l)
        for l in range(NBUF):
            select(pv, l + NBUF, l, sg * L + l + NBUF)
        return carry

    lax.fori_loop(0, BPW // L, step, 0)

    for h in range(2):
        pltpu.sync_copy(acc.at[h], out_hbm.at[:, pl.ds(base + h * 128, 128)])


def kernel(x, token_table, pos_table):
    batch, seq = x.shape
    idx = x.astype(jnp.int32).reshape(NW * 2, 128)
    out_t = _emb_lookup(idx, token_table.T, pos_table.T)
    return out_t.T.reshape(batch, seq, D)


# speedup vs baseline: 5.3837x; 1.1136x over previous
import functools

import jax
import jax.numpy as jnp
from jax import lax
from jax.experimental import pallas as pl
from jax.experimental.pallas import tpu as pltpu
from jax.experimental.pallas import tpu_sc as plsc

D = 64
SEQ = 2048
B_TOTAL = 8192
NC, NS, L = 2, 16, 16
NW = NC * NS
BPW = B_TOTAL // NW
GRP = 4
NGRP = BPW // GRP
NSG = BPW // L
NBUF = 12

_mesh = plsc.VectorSubcoreMesh(core_axis_name="c", subcore_axis_name="s")


@functools.partial(
    pl.kernel,
    mesh=_mesh,
    out_type=jax.ShapeDtypeStruct((D, B_TOTAL), jnp.float32),
    scratch_types=[
        pltpu.VMEM((8, 128), jnp.int32),
        pltpu.VMEM((NBUF, D, 128), jnp.float32),
        pltpu.VMEM((2, D, 128), jnp.float32),
        pltpu.SemaphoreType.DMA,
        pltpu.SemaphoreType.DMA,
    ],
    compiler_params=pltpu.CompilerParams(needs_layout_passes=False),
)
def _emb_lookup(idx_hbm, tok_hbm, pos_hbm, out_hbm,
                idx_v, stage, acc, sem, psem):
    wid = lax.axis_index("s") * NC + lax.axis_index("c")
    base = pl.multiple_of(wid * BPW, BPW)
    pos_base = pl.multiple_of(lax.rem(base, SEQ), BPW)

    blk0 = pl.multiple_of((wid // 4) * 8, 8)
    pltpu.sync_copy(idx_hbm.at[pl.ds(blk0, 8)], idx_v)
    pos_cps = [
        pltpu.async_copy(
            pos_hbm.at[:, pl.ds(pos_base + h * 128, 128)], acc.at[h], psem
        )
        for h in range(2)
    ]

    row0 = lax.rem(wid, 4) * 2
    lanes = lax.iota(jnp.int32, L)

    def load_iv(sg):
        sgc = lax.min(sg, NSG - 1)
        return idx_v[row0 + sgc // 8, pl.ds(lax.rem(sgc, 8) * L, L)]

    def fire(cv, l, r):
        col = pl.multiple_of(lax.shift_right_logical(cv[l], 7) * 128, 128)
        pltpu.async_copy(
            tok_hbm.at[:, pl.ds(col, 128)], stage.at[lax.rem(r, NBUF)], sem
        )

    def fire_group(cv, lb, g):
        if isinstance(g, int):
            for l in range(GRP):
                fire(cv, lb + l, g * GRP + l)
            return

        @pl.when(g < NGRP)
        def _():
            for l in range(GRP):
                fire(cv, lb + l, g * GRP + l)

    def select(pv, l, r):
        slot = lax.rem(r, NBUF)
        pltpu.make_async_copy(
            tok_hbm.at[:, pl.ds(0, 128)], stage.at[slot], sem
        ).wait()
        pvec = jnp.full((L,), lax.bitwise_and(pv[l], 127), jnp.int32)
        rvec = jnp.full((L,), lax.rem(r, 128), jnp.int32)
        buf = stage.at[slot]
        half = acc.at[r // 128]
        for q in range(D // L):
            dvec = lanes + (q * L)
            vals = plsc.load_gather(buf, [dvec, pvec])
            plsc.addupdate_scatter(half, [dvec, rvec], vals)

    iv0 = load_iv(0)
    for g in range(3):
        fire_group(iv0, g * GRP, g)
    for cp in pos_cps:
        cp.wait()

    def step(sg, iv):
        iv_next = load_iv(sg + 1)
        for j in range(4):
            g_sel = sg * 4 + j
            for l in range(GRP):
                select(iv, j * GRP + l, g_sel * GRP + l)
            if j == 0:
                fire_group(iv, 3 * GRP, g_sel + 3)
            else:
                fire_group(iv_next, (j - 1) * GRP, g_sel + 3)
        return iv_next

    lax.fori_loop(0, NSG, step, iv0)

    for h in range(2):
        pltpu.sync_copy(acc.at[h], out_hbm.at[:, pl.ds(base + h * 128, 128)])


def kernel(x, token_table, pos_table):
    batch, seq = x.shape
    idx = x.astype(jnp.int32).reshape(NW * 2, 128)
    out_t = _emb_lookup(idx, token_table.T, pos_table.T)
    return out_t.T.reshape(batch, seq, D)
